# baseline (speedup 1.0000x reference)
import jax
import jax.numpy as jnp
from jax import lax
from jax.experimental import pallas as pl
from jax.experimental.pallas import tpu as pltpu
from jax.experimental.pallas import tpu_sc as plsc

N = 10000
E = 320000
IN_DIM = 128
H = 8
D = 16
NEG_SLOPE = 0.2

NC = 2
NS = 16
NW = NC * NS
CHUNK = 80
NCHUNKS = E // CHUNK
OUTER = NCHUNKS // NW
RPT = N // NS
ZR = 125

_mesh = plsc.VectorSubcoreMesh(core_axis_name="c", subcore_axis_name="s")
_sc_params = pltpu.CompilerParams(use_tc_tiling_on_sc=False)


def _feat_body(h_ref, w_ref, al_ref, ar_ref, feat_ref, el_ref, er_ref):
    feat = jnp.dot(h_ref[...], w_ref[...], preferred_element_type=jnp.float32)
    r = lax.broadcasted_iota(jnp.int32, (IN_DIM, 16), 0) // D
    c = lax.broadcasted_iota(jnp.int32, (IN_DIM, 16), 1)
    sel = (r == c).astype(jnp.float32)
    el_ref[...] = jnp.dot(feat * al_ref[...], sel, preferred_element_type=jnp.float32)
    er_ref[...] = jnp.dot(feat * ar_ref[...], sel, preferred_element_type=jnp.float32)
    feat_ref[...] = feat


_BN = 400
_feat_call = pl.pallas_call(
    _feat_body,
    grid=(N // _BN,),
    in_specs=[
        pl.BlockSpec((_BN, IN_DIM), lambda i: (i, 0)),
        pl.BlockSpec((IN_DIM, IN_DIM), lambda i: (0, 0)),
        pl.BlockSpec((1, IN_DIM), lambda i: (0, 0)),
        pl.BlockSpec((1, IN_DIM), lambda i: (0, 0)),
    ],
    out_specs=[
        pl.BlockSpec((_BN, IN_DIM), lambda i: (i, 0)),
        pl.BlockSpec((_BN, 16), lambda i: (i, 0)),
        pl.BlockSpec((_BN, 16), lambda i: (i, 0)),
    ],
    out_shape=[
        jax.ShapeDtypeStruct((N, IN_DIM), jnp.float32),
        jax.ShapeDtypeStruct((N, 16), jnp.float32),
        jax.ShapeDtypeStruct((N, 16), jnp.float32),
    ],
)


def _edge_body(feat_hbm, el_hbm, er_hbm, src_hbm, dst_hbm,
               part_hbm, partw_hbm,
               acc, accw,
               srcv0, dstv0, dstsc0, frows0, elrows0, errows0,
               srcv1, dstv1, dstsc1, frows1, elrows1, errows1,
               zbuf, zbufw,
               isem0, gsemf0, gseme0, gsemr0, ssemf0, ssemw0,
               isem1, gsemf1, gseme1, gsemr1, ssemf1, ssemw1,
               zsem):
    c = lax.axis_index("c")
    s = lax.axis_index("s")
    wid = c * NS + s

    slots = (
        (srcv0, dstv0, dstsc0, frows0, elrows0, errows0,
         isem0, gsemf0, gseme0, gsemr0, ssemf0, ssemw0),
        (srcv1, dstv1, dstsc1, frows1, elrows1, errows1,
         isem1, gsemf1, gseme1, gsemr1, ssemf1, ssemw1),
    )

    def idx_issue(i, b):
        srcv, dstv = slots[b][0], slots[b][1]
        isem = slots[b][6]
        base = (i * NW + wid) * CHUNK
        pltpu.async_copy(src_hbm.at[pl.ds(base, CHUNK)], srcv, isem)
        pltpu.async_copy(dst_hbm.at[pl.ds(base, CHUNK)], dstv, isem)

    def idx_wait(b):
        srcv, dstv = slots[b][0], slots[b][1]
        isem = slots[b][6]
        pltpu.make_async_copy(src_hbm.at[pl.ds(0, CHUNK)], srcv, isem).wait()
        pltpu.make_async_copy(dst_hbm.at[pl.ds(0, CHUNK)], dstv, isem).wait()

    def gather_issue(b):
        srcv, dstv, _, frows, elrows, errows = slots[b][:6]
        gsemf, gseme, gsemr = slots[b][7:10]
        pltpu.async_copy(feat_hbm.at[srcv], frows, gsemf)
        pltpu.async_copy(el_hbm.at[srcv], elrows, gseme)
        pltpu.async_copy(er_hbm.at[dstv], errows, gsemr)

    def gather_wait(b):
        srcv, dstv, _, frows, elrows, errows = slots[b][:6]
        gsemf, gseme, gsemr = slots[b][7:10]
        pltpu.make_async_copy(feat_hbm.at[srcv], frows, gsemf).wait()
        pltpu.make_async_copy(el_hbm.at[srcv], elrows, gseme).wait()
        pltpu.make_async_copy(er_hbm.at[dstv], errows, gsemr).wait()

    def scatter_issue(b):
        dstsc, frows, _, errows = slots[b][2], slots[b][3], slots[b][4], slots[b][5]
        ssemf, ssemw = slots[b][10], slots[b][11]
        pltpu.async_copy(frows, acc.at[dstsc], ssemf, add=True)
        pltpu.async_copy(errows, accw.at[dstsc], ssemw, add=True)

    def scatter_wait(b):
        dstsc, frows, _, errows = slots[b][2], slots[b][3], slots[b][4], slots[b][5]
        ssemf, ssemw = slots[b][10], slots[b][11]
        pltpu.make_async_copy(frows, acc.at[dstsc], ssemf).wait()
        pltpu.make_async_copy(errows, accw.at[dstsc], ssemw).wait()

    def dst_copy(b):
        dstv, dstsc = slots[b][1], slots[b][2]
        for j in range(CHUNK // 16):
            dstsc[pl.ds(j * 16, 16)] = dstv[pl.ds(j * 16, 16)]

    def compute(b):
        dstv, dstsc, frows, elrows, errows = slots[b][1:6]

        def wbody(j, carry):
            x = elrows[j, :] + errows[j, :]
            errows[j, :] = jnp.exp(jnp.where(x > 0, x, x * NEG_SLOPE))
            return carry

        lax.fori_loop(0, CHUNK, wbody, 0, unroll=4)

        def sbody(e, carry):
            wv = errows[e, :]
            for hh in range(H):
                fh = frows[e, pl.ds(hh * D, D)]
                frows[e, pl.ds(hh * D, D)] = fh * wv[hh]
            return carry

        lax.fori_loop(0, CHUNK, sbody, 0, unroll=2)

    zero = jnp.zeros((16,), jnp.float32)

    def zrow(r, carry):
        for j in range(IN_DIM // 16):
            zbuf[r, pl.ds(j * 16, 16)] = zero
        zbufw[r, :] = zero
        return carry

    lax.fori_loop(0, ZR, zrow, 0)
    row0 = s * RPT
    for k in range(RPT // 25):
        pltpu.async_copy(zbuf.at[pl.ds(0, 25)], acc.at[pl.ds(row0 + k * 25, 25)], zsem)
    for k in range(RPT // ZR):
        pltpu.async_copy(zbufw, accw.at[pl.ds(row0 + k * ZR, ZR)], zsem)
    for k in range(RPT // 25):
        pltpu.make_async_copy(zbuf.at[pl.ds(0, 25)], acc.at[pl.ds(row0 + k * 25, 25)], zsem).wait()
    for k in range(RPT // ZR):
        pltpu.make_async_copy(zbufw, accw.at[pl.ds(row0 + k * ZR, ZR)], zsem).wait()
    plsc.subcore_barrier()

    def step_prefetch(nb):
        scatter_wait(nb)
        idx_wait(nb)
        gather_issue(nb)

    def step_compute(i, b, issue_next_idx):
        gather_wait(b)
        dst_copy(b)
        if issue_next_idx:
            idx_issue(i + 2, b)
        compute(b)
        scatter_issue(b)

    def fetch_issue(i, b):
        srcv, dstv, _, frows, elrows, errows = slots[b][:6]
        gsemf, gseme, gsemr = slots[b][7:10]
        idx_issue(i, b)
        idx_wait(b)
        pltpu.async_copy(feat_hbm.at[srcv], frows, gsemf)
        pltpu.async_copy(el_hbm.at[srcv], elrows, gseme)
        pltpu.async_copy(er_hbm.at[dstv], errows, gsemr)

    def wloop(b):
        srcv, dstv, _, frows, elrows, errows = slots[b][:6]
        gsemf, gseme, gsemr = slots[b][7:10]
        pltpu.make_async_copy(el_hbm.at[srcv], elrows, gseme).wait()
        pltpu.make_async_copy(er_hbm.at[dstv], errows, gsemr).wait()
        dst_copy(b)

        def wbody(j, wcarry):
            x = elrows[j, :] + errows[j, :]
            errows[j, :] = jnp.exp(jnp.where(x > 0, x, x * NEG_SLOPE))
            return wcarry

        lax.fori_loop(0, CHUNK, wbody, 0, unroll=4)
        pltpu.make_async_copy(feat_hbm.at[srcv], frows, gsemf).wait()

    def scale(b):
        frows, errows = slots[b][3], slots[b][5]

        def sbody(e, scarry):
            wv = errows[e, :]
            for hh in range(H):
                fh = frows[e, pl.ds(hh * D, D)]
                frows[e, pl.ds(hh * D, D)] = fh * wv[hh]
            return scarry

        lax.fori_loop(0, CHUNK, sbody, 0, unroll=2)

    def chunk_body(i, carry):
        fetch_issue(i, 0)
        wloop(0)
        scale(0)
        scatter_issue(0)
        scatter_wait(0)
        return carry

    lax.fori_loop(0, OUTER, chunk_body, 0)
    plsc.subcore_barrier()

    for k in range(RPT // ZR):
        r0 = row0 + k * ZR
        pltpu.async_copy(acc.at[pl.ds(r0, ZR)], part_hbm.at[c, pl.ds(r0, ZR)], zsem)
        pltpu.async_copy(accw.at[pl.ds(r0, ZR)], partw_hbm.at[c, pl.ds(r0, ZR)], zsem)
    for k in range(RPT // ZR):
        r0 = row0 + k * ZR
        pltpu.make_async_copy(acc.at[pl.ds(r0, ZR)], part_hbm.at[c, pl.ds(r0, ZR)], zsem).wait()
        pltpu.make_async_copy(accw.at[pl.ds(r0, ZR)], partw_hbm.at[c, pl.ds(r0, ZR)], zsem).wait()


_edge_call = pl.kernel(
    _edge_body,
    out_type=[
        jax.ShapeDtypeStruct((NC, N, IN_DIM), jnp.float32),
        jax.ShapeDtypeStruct((NC, N, 16), jnp.float32),
    ],
    mesh=_mesh,
    scratch_types=[
        pltpu.VMEM_SHARED((N, IN_DIM), jnp.float32),
        pltpu.VMEM_SHARED((N, 16), jnp.float32),
    ] + 2 * [
        pltpu.VMEM((CHUNK,), jnp.int32),
        pltpu.VMEM((CHUNK,), jnp.int32),
        pltpu.VMEM((CHUNK,), jnp.int32),
        pltpu.VMEM((CHUNK, IN_DIM), jnp.float32),
        pltpu.VMEM((CHUNK, 16), jnp.float32),
        pltpu.VMEM((CHUNK, 16), jnp.float32),
    ] + [
        pltpu.VMEM((25, IN_DIM), jnp.float32),
        pltpu.VMEM((ZR, 16), jnp.float32),
    ] + 13 * [pltpu.SemaphoreType.DMA],
    compiler_params=_sc_params,
)


def _fin_body(p_ref, w_ref, h_ref, b_ref, o_ref):
    acc = p_ref[0] + p_ref[1]
    sw = w_ref[0] + w_ref[1]
    r = lax.broadcasted_iota(jnp.int32, (16, IN_DIM), 0)
    c = lax.broadcasted_iota(jnp.int32, (16, IN_DIM), 1) // D
    sel_t = (r == c).astype(jnp.float32)
    sexp = jnp.dot(sw, sel_t, preferred_element_type=jnp.float32)
    rst = jnp.where(sexp > 0, acc / jnp.where(sexp > 0, sexp, 1.0), 0.0)
    z = rst + b_ref[...]
    o_ref[...] = h_ref[...] + jnp.where(z > 0, z, jnp.exp(z) - 1.0)


_finish_call = pl.pallas_call(
    _fin_body,
    grid=(N // _BN,),
    in_specs=[
        pl.BlockSpec((NC, _BN, IN_DIM), lambda i: (0, i, 0)),
        pl.BlockSpec((NC, _BN, 16), lambda i: (0, i, 0)),
        pl.BlockSpec((_BN, IN_DIM), lambda i: (i, 0)),
        pl.BlockSpec((1, IN_DIM), lambda i: (0, 0)),
    ],
    out_specs=pl.BlockSpec((_BN, IN_DIM), lambda i: (i, 0)),
    out_shape=jax.ShapeDtypeStruct((N, IN_DIM), jnp.float32),
)


def kernel(h, edge_index, W, attn_l, attn_r, bias):
    feat, el16, er16 = _feat_call(
        h, W, attn_l.reshape(1, IN_DIM), attn_r.reshape(1, IN_DIM)
    )
    src = edge_index[0]
    dst = edge_index[1]
    part, partw = _edge_call(feat, el16, er16, src, dst)
    return _finish_call(part, partw, h, bias.reshape(1, IN_DIM))


# speedup vs baseline: 1.0002x; 1.0002x over previous
import jax
import jax.numpy as jnp
from jax import lax
from jax.experimental import pallas as pl
from jax.experimental.pallas import tpu as pltpu
from jax.experimental.pallas import tpu_sc as plsc

N = 10000
E = 320000
IN_DIM = 128
H = 8
D = 16
NEG_SLOPE = 0.2

NC = 2
NS = 16
NW = NC * NS
CHUNK = 80
NCHUNKS = E // CHUNK
OUTER = NCHUNKS // NW
RPT = N // NS
ZR = 125

_mesh = plsc.VectorSubcoreMesh(core_axis_name="c", subcore_axis_name="s")
_sc_params = pltpu.CompilerParams(use_tc_tiling_on_sc=False)


def _feat_body(h_ref, w_ref, al_ref, ar_ref, feat_ref, el_ref, er_ref):
    feat = jnp.dot(h_ref[...], w_ref[...], preferred_element_type=jnp.float32)
    r = lax.broadcasted_iota(jnp.int32, (IN_DIM, 16), 0) // D
    c = lax.broadcasted_iota(jnp.int32, (IN_DIM, 16), 1)
    sel = (r == c).astype(jnp.float32)
    el_ref[...] = jnp.dot(feat * al_ref[...], sel, preferred_element_type=jnp.float32)
    er_ref[...] = jnp.dot(feat * ar_ref[...], sel, preferred_element_type=jnp.float32)
    feat_ref[...] = feat


_BN = 400
_feat_call = pl.pallas_call(
    _feat_body,
    grid=(N // _BN,),
    in_specs=[
        pl.BlockSpec((_BN, IN_DIM), lambda i: (i, 0)),
        pl.BlockSpec((IN_DIM, IN_DIM), lambda i: (0, 0)),
        pl.BlockSpec((1, IN_DIM), lambda i: (0, 0)),
        pl.BlockSpec((1, IN_DIM), lambda i: (0, 0)),
    ],
    out_specs=[
        pl.BlockSpec((_BN, IN_DIM), lambda i: (i, 0)),
        pl.BlockSpec((_BN, 16), lambda i: (i, 0)),
        pl.BlockSpec((_BN, 16), lambda i: (i, 0)),
    ],
    out_shape=[
        jax.ShapeDtypeStruct((N, IN_DIM), jnp.float32),
        jax.ShapeDtypeStruct((N, 16), jnp.float32),
        jax.ShapeDtypeStruct((N, 16), jnp.float32),
    ],
)


def _edge_body(feat_hbm, el_hbm, er_hbm, src_hbm, dst_hbm,
               part_hbm, partw_hbm,
               acc, accw,
               srcv0, dstv0, dstsc0, frows0, elrows0, errows0,
               srcv1, dstv1, dstsc1, frows1, elrows1, errows1,
               zbuf, zbufw,
               isem0, gsemf0, gseme0, gsemr0, ssemf0, ssemw0,
               isem1, gsemf1, gseme1, gsemr1, ssemf1, ssemw1,
               zsem):
    c = lax.axis_index("c")
    s = lax.axis_index("s")
    wid = c * NS + s

    slots = (
        (srcv0, dstv0, dstsc0, frows0, elrows0, errows0,
         isem0, gsemf0, gseme0, gsemr0, ssemf0, ssemw0),
        (srcv1, dstv1, dstsc1, frows1, elrows1, errows1,
         isem1, gsemf1, gseme1, gsemr1, ssemf1, ssemw1),
    )

    def idx_issue(i, b):
        srcv, dstv = slots[b][0], slots[b][1]
        isem = slots[b][6]
        base = (i * NW + wid) * CHUNK
        pltpu.async_copy(src_hbm.at[pl.ds(base, CHUNK)], srcv, isem)
        pltpu.async_copy(dst_hbm.at[pl.ds(base, CHUNK)], dstv, isem)

    def idx_wait(b):
        srcv, dstv = slots[b][0], slots[b][1]
        isem = slots[b][6]
        pltpu.make_async_copy(src_hbm.at[pl.ds(0, CHUNK)], srcv, isem).wait()
        pltpu.make_async_copy(dst_hbm.at[pl.ds(0, CHUNK)], dstv, isem).wait()

    def scatter_issue(b):
        dstsc, frows, _, errows = slots[b][2], slots[b][3], slots[b][4], slots[b][5]
        ssemf, ssemw = slots[b][10], slots[b][11]
        pltpu.async_copy(frows, acc.at[dstsc], ssemf, add=True)
        pltpu.async_copy(errows, accw.at[dstsc], ssemw, add=True)

    def scatter_wait(b):
        dstsc, frows, _, errows = slots[b][2], slots[b][3], slots[b][4], slots[b][5]
        ssemf, ssemw = slots[b][10], slots[b][11]
        pltpu.make_async_copy(frows, acc.at[dstsc], ssemf).wait()
        pltpu.make_async_copy(errows, accw.at[dstsc], ssemw).wait()

    def dst_copy(b):
        dstv, dstsc = slots[b][1], slots[b][2]
        for j in range(CHUNK // 16):
            dstsc[pl.ds(j * 16, 16)] = dstv[pl.ds(j * 16, 16)]

    zero = jnp.zeros((16,), jnp.float32)

    def zrow(r, carry):
        for j in range(IN_DIM // 16):
            zbuf[r, pl.ds(j * 16, 16)] = zero
        zbufw[r, :] = zero
        return carry

    lax.fori_loop(0, ZR, zrow, 0)
    row0 = s * RPT
    for k in range(RPT // 25):
        pltpu.async_copy(zbuf.at[pl.ds(0, 25)], acc.at[pl.ds(row0 + k * 25, 25)], zsem)
    for k in range(RPT // ZR):
        pltpu.async_copy(zbufw, accw.at[pl.ds(row0 + k * ZR, ZR)], zsem)
    for k in range(RPT // 25):
        pltpu.make_async_copy(zbuf.at[pl.ds(0, 25)], acc.at[pl.ds(row0 + k * 25, 25)], zsem).wait()
    for k in range(RPT // ZR):
        pltpu.make_async_copy(zbufw, accw.at[pl.ds(row0 + k * ZR, ZR)], zsem).wait()
    plsc.subcore_barrier()

    def fetch_issue(i, b):
        srcv, dstv, _, frows, elrows, errows = slots[b][:6]
        gsemf, gseme, gsemr = slots[b][7:10]
        idx_issue(i, b)
        idx_wait(b)
        pltpu.async_copy(feat_hbm.at[srcv], frows, gsemf)
        pltpu.async_copy(el_hbm.at[srcv], elrows, gseme)
        pltpu.async_copy(er_hbm.at[dstv], errows, gsemr)

    def wloop(b):
        srcv, dstv, _, frows, elrows, errows = slots[b][:6]
        gsemf, gseme, gsemr = slots[b][7:10]
        pltpu.make_async_copy(el_hbm.at[srcv], elrows, gseme).wait()
        pltpu.make_async_copy(er_hbm.at[dstv], errows, gsemr).wait()
        dst_copy(b)

        def wbody(j, wcarry):
            x = elrows[j, :] + errows[j, :]
            errows[j, :] = jnp.exp(jnp.where(x > 0, x, x * NEG_SLOPE))
            return wcarry

        lax.fori_loop(0, CHUNK, wbody, 0, unroll=4)
        pltpu.make_async_copy(feat_hbm.at[srcv], frows, gsemf).wait()

    def scale(b):
        frows, errows = slots[b][3], slots[b][5]

        def sbody(e, scarry):
            wv = errows[e, :]
            for hh in range(H):
                fh = frows[e, pl.ds(hh * D, D)]
                frows[e, pl.ds(hh * D, D)] = fh * wv[hh]
            return scarry

        lax.fori_loop(0, CHUNK, sbody, 0, unroll=2)

    def chunk_body(i, carry):
        fetch_issue(i, 0)
        wloop(0)
        scale(0)
        scatter_issue(0)
        scatter_wait(0)
        return carry

    lax.fori_loop(0, OUTER, chunk_body, 0)
    plsc.subcore_barrier()

    for k in range(RPT // ZR):
        r0 = row0 + k * ZR
        pltpu.async_copy(acc.at[pl.ds(r0, ZR)], part_hbm.at[c, pl.ds(r0, ZR)], zsem)
        pltpu.async_copy(accw.at[pl.ds(r0, ZR)], partw_hbm.at[c, pl.ds(r0, ZR)], zsem)
    for k in range(RPT // ZR):
        r0 = row0 + k * ZR
        pltpu.make_async_copy(acc.at[pl.ds(r0, ZR)], part_hbm.at[c, pl.ds(r0, ZR)], zsem).wait()
        pltpu.make_async_copy(accw.at[pl.ds(r0, ZR)], partw_hbm.at[c, pl.ds(r0, ZR)], zsem).wait()


_edge_call = pl.kernel(
    _edge_body,
    out_type=[
        jax.ShapeDtypeStruct((NC, N, IN_DIM), jnp.float32),
        jax.ShapeDtypeStruct((NC, N, 16), jnp.float32),
    ],
    mesh=_mesh,
    scratch_types=[
        pltpu.VMEM_SHARED((N, IN_DIM), jnp.float32),
        pltpu.VMEM_SHARED((N, 16), jnp.float32),
    ] + 2 * [
        pltpu.VMEM((CHUNK,), jnp.int32),
        pltpu.VMEM((CHUNK,), jnp.int32),
        pltpu.VMEM((CHUNK,), jnp.int32),
        pltpu.VMEM((CHUNK, IN_DIM), jnp.float32),
        pltpu.VMEM((CHUNK, 16), jnp.float32),
        pltpu.VMEM((CHUNK, 16), jnp.float32),
    ] + [
        pltpu.VMEM((25, IN_DIM), jnp.float32),
        pltpu.VMEM((ZR, 16), jnp.float32),
    ] + 13 * [pltpu.SemaphoreType.DMA],
    compiler_params=_sc_params,
)


def _fin_body(p_ref, w_ref, h_ref, b_ref, o_ref):
    acc = p_ref[0] + p_ref[1]
    sw = w_ref[0] + w_ref[1]
    r = lax.broadcasted_iota(jnp.int32, (16, IN_DIM), 0)
    c = lax.broadcasted_iota(jnp.int32, (16, IN_DIM), 1) // D
    sel_t = (r == c).astype(jnp.float32)
    sexp = jnp.dot(sw, sel_t, preferred_element_type=jnp.float32)
    rst = jnp.where(sexp > 0, acc / jnp.where(sexp > 0, sexp, 1.0), 0.0)
    z = rst + b_ref[...]
    o_ref[...] = h_ref[...] + jnp.where(z > 0, z, jnp.exp(z) - 1.0)


_finish_call = pl.pallas_call(
    _fin_body,
    grid=(N // _BN,),
    in_specs=[
        pl.BlockSpec((NC, _BN, IN_DIM), lambda i: (0, i, 0)),
        pl.BlockSpec((NC, _BN, 16), lambda i: (0, i, 0)),
        pl.BlockSpec((_BN, IN_DIM), lambda i: (i, 0)),
        pl.BlockSpec((1, IN_DIM), lambda i: (0, 0)),
    ],
    out_specs=pl.BlockSpec((_BN, IN_DIM), lambda i: (i, 0)),
    out_shape=jax.ShapeDtypeStruct((N, IN_DIM), jnp.float32),
)


def kernel(h, edge_index, W, attn_l, attn_r, bias):
    feat, el16, er16 = _feat_call(
        h, W, attn_l.reshape(1, IN_DIM), attn_r.reshape(1, IN_DIM)
    )
    src = edge_index[0]
    dst = edge_index[1]
    part, partw = _edge_call(feat, el16, er16, src, dst)
    return _finish_call(part, partw, h, bias.reshape(1, IN_DIM))
